# Initial kernel scaffold; baseline (speedup 1.0000x reference)
import jax
import jax.numpy as jnp
from jax.experimental import pallas as pl


def kernel(x, index):
    raise NotImplementedError("write your pallas kernel here")


# speedup vs baseline: 3.7902x; 3.7902x over previous
import functools
import math

import jax
import jax.numpy as jnp
from jax import lax
from jax.experimental import pallas as pl
from jax.experimental.pallas import tpu as pltpu
from jax.experimental.pallas import tpu_sc as plsc

_AVG_AGG = 32.0
_SCALE = 1.0 / math.sqrt(_AVG_AGG)
_N = 10000
_NC = 2
_NS = 16
_NW = _NC * _NS
_CHUNK = 80


def _sc_scatter_partials(x, idx):
    e_total, d = x.shape
    assert d == 128
    assert e_total % (_NW * _CHUNK) == 0
    e_per_w = e_total // _NW
    n_chunks = e_per_w // _CHUNK
    rows_per_tile = (_N // _NS) // 8 * 8
    rows_rem = _N - rows_per_tile * _NS

    mesh = plsc.VectorSubcoreMesh(core_axis_name="c", subcore_axis_name="s")

    @functools.partial(
        pl.kernel,
        mesh=mesh,
        out_type=jax.ShapeDtypeStruct((_NC, _N, d), jnp.float32),
        scratch_types=[
            pltpu.VMEM((_CHUNK,), jnp.int32),
            pltpu.VMEM((_CHUNK, d), jnp.float32),
            pltpu.VMEM_SHARED((_N, d), jnp.float32),
        ],
    )
    def scatter_kernel(x_hbm, idx_hbm, out_hbm, idx_v, rows_v, acc_sh):
        cid = lax.axis_index("c")
        sid = lax.axis_index("s")
        wid = cid * _NS + sid

        zeros16 = jnp.zeros((16,), jnp.float32)

        def zero_row(i, _):
            for j in range(d // 16):
                rows_v[i, pl.ds(j * 16, 16)] = zeros16
            return 0

        lax.fori_loop(0, _CHUNK, zero_row, 0)

        zbase = sid * rows_per_tile
        full, rem = divmod(rows_per_tile, _CHUNK)
        for k in range(full):
            pltpu.sync_copy(rows_v, acc_sh.at[pl.ds(zbase + k * _CHUNK, _CHUNK)])
        if rem:
            pltpu.sync_copy(
                rows_v.at[pl.ds(0, rem)],
                acc_sh.at[pl.ds(zbase + full * _CHUNK, rem)],
            )

        @pl.when(sid == 0)
        def _zero_tail():
            pltpu.sync_copy(
                rows_v.at[pl.ds(0, rows_rem)],
                acc_sh.at[pl.ds(rows_per_tile * _NS, rows_rem)],
            )

        plsc.subcore_barrier()

        ebase = wid * e_per_w

        def chunk_body(c, _):
            b = ebase + c * _CHUNK
            pltpu.sync_copy(idx_hbm.at[pl.ds(b, _CHUNK)], idx_v)
            pltpu.sync_copy(x_hbm.at[pl.ds(b, _CHUNK)], rows_v)
            pltpu.sync_copy(rows_v, acc_sh.at[idx_v], add=True)
            return 0

        lax.fori_loop(0, n_chunks, chunk_body, 0)
        plsc.subcore_barrier()

        obase = sid * rows_per_tile
        pltpu.sync_copy(
            acc_sh.at[pl.ds(obase, rows_per_tile)],
            out_hbm.at[cid, pl.ds(obase, rows_per_tile)],
        )

        @pl.when(sid == 0)
        def _write_tail():
            pltpu.sync_copy(
                acc_sh.at[pl.ds(rows_per_tile * _NS, rows_rem)],
                out_hbm.at[cid, pl.ds(rows_per_tile * _NS, rows_rem)],
            )

    return scatter_kernel(x, idx)


def _combine(p_ref, o_ref):
    o_ref[...] = (p_ref[0] + p_ref[1]) * _SCALE


def kernel(x, index):
    idx = index.astype(jnp.int32)
    partials = _sc_scatter_partials(x, idx)
    n, d = _N, x.shape[1]
    blk = 1000
    out = pl.pallas_call(
        _combine,
        grid=(n // blk,),
        in_specs=[pl.BlockSpec((_NC, blk, d), lambda i: (0, i, 0))],
        out_specs=pl.BlockSpec((blk, d), lambda i: (i, 0)),
        out_shape=jax.ShapeDtypeStruct((n, d), jnp.float32),
    )(partials)
    return out


# speedup vs baseline: 9.0398x; 2.3850x over previous
import functools
import math

import jax
import jax.numpy as jnp
from jax import lax
from jax.experimental import pallas as pl
from jax.experimental.pallas import tpu as pltpu
from jax.experimental.pallas import tpu_sc as plsc

_AVG_AGG = 32.0
_SCALE = 1.0 / math.sqrt(_AVG_AGG)
_N = 10000
_NC = 2
_NS = 16
_NW = _NC * _NS
_B = 80
_RING = 4


def _sc_scatter_partials(x, idx):
    e_total, d = x.shape
    assert d == 128
    assert e_total % (_NW * _B) == 0
    e_per_w = e_total // _NW
    n_iter = e_per_w // _B
    assert n_iter % _RING == 1
    rows_per_tile = (_N // _NS) // 8 * 8
    rows_rem = _N - rows_per_tile * _NS

    mesh = plsc.VectorSubcoreMesh(core_axis_name="c", subcore_axis_name="s")

    @functools.partial(
        pl.kernel,
        mesh=mesh,
        out_type=jax.ShapeDtypeStruct((_NC, _N, d), jnp.float32),
        scratch_types=(
            [pltpu.VMEM((_B,), jnp.int32) for _ in range(_RING)]
            + [pltpu.VMEM((_B, d), jnp.float32) for _ in range(_RING)]
            + [pltpu.VMEM_SHARED((_N, d), jnp.float32)]
            + [pltpu.SemaphoreType.DMA for _ in range(_RING)]
        ),
    )
    def scatter_kernel(x_hbm, idx_hbm, out_hbm, *refs):
        idx_v = refs[:_RING]
        rows_v = refs[_RING:2 * _RING]
        acc_sh = refs[2 * _RING]
        sems = refs[2 * _RING + 1:]

        cid = lax.axis_index("c")
        sid = lax.axis_index("s")
        wid = cid * _NS + sid

        zeros16 = jnp.zeros((16,), jnp.float32)

        def zero_row(i, _):
            for j in range(d // 16):
                rows_v[0][i, pl.ds(j * 16, 16)] = zeros16
            return 0

        lax.fori_loop(0, _B, zero_row, 0)

        zbase = sid * rows_per_tile
        full, rem = divmod(rows_per_tile, _B)
        for k in range(full):
            pltpu.sync_copy(rows_v[0], acc_sh.at[pl.ds(zbase + k * _B, _B)])
        if rem:
            pltpu.sync_copy(
                rows_v[0].at[pl.ds(0, rem)],
                acc_sh.at[pl.ds(zbase + full * _B, rem)],
            )

        @pl.when(sid == 0)
        def _zero_tail():
            pltpu.sync_copy(
                rows_v[0].at[pl.ds(0, rows_rem)],
                acc_sh.at[pl.ds(rows_per_tile * _NS, rows_rem)],
            )

        plsc.subcore_barrier()

        ebase = wid * e_per_w

        def start_loads(g, b):
            pltpu.async_copy(idx_hbm.at[pl.ds(ebase + g * _B, _B)],
                             idx_v[b], sems[b])
            pltpu.async_copy(x_hbm.at[pl.ds(ebase + g * _B, _B)],
                             rows_v[b], sems[b])

        def wait_loads(g, b):
            pltpu.make_async_copy(idx_hbm.at[pl.ds(ebase + g * _B, _B)],
                                  idx_v[b], sems[b]).wait()
            pltpu.make_async_copy(x_hbm.at[pl.ds(ebase + g * _B, _B)],
                                  rows_v[b], sems[b]).wait()

        for b in range(_RING):
            start_loads(b, b)

        @pl.loop(0, n_iter - 1, step=_RING)
        def _ring(i):
            for b in range(_RING):
                g = i + b
                wait_loads(g, b)
                pltpu.sync_copy(rows_v[b], acc_sh.at[idx_v[b]], add=True)

                @pl.when(g + _RING < n_iter)
                def _prefetch():
                    start_loads(g + _RING, b)

        g_last = n_iter - 1
        wait_loads(g_last, 0)
        pltpu.sync_copy(rows_v[0], acc_sh.at[idx_v[0]], add=True)
        plsc.subcore_barrier()

        obase = sid * rows_per_tile
        pltpu.sync_copy(
            acc_sh.at[pl.ds(obase, rows_per_tile)],
            out_hbm.at[cid, pl.ds(obase, rows_per_tile)],
        )

        @pl.when(sid == 0)
        def _write_tail():
            pltpu.sync_copy(
                acc_sh.at[pl.ds(rows_per_tile * _NS, rows_rem)],
                out_hbm.at[cid, pl.ds(rows_per_tile * _NS, rows_rem)],
            )

    return scatter_kernel(x, idx)


def _combine(p_ref, o_ref):
    o_ref[...] = (p_ref[0] + p_ref[1]) * _SCALE


def kernel(x, index):
    idx = index.astype(jnp.int32)
    partials = _sc_scatter_partials(x, idx)
    n, d = _N, x.shape[1]
    blk = 1000
    out = pl.pallas_call(
        _combine,
        grid=(n // blk,),
        in_specs=[pl.BlockSpec((_NC, blk, d), lambda i: (0, i, 0))],
        out_specs=pl.BlockSpec((blk, d), lambda i: (i, 0)),
        out_shape=jax.ShapeDtypeStruct((n, d), jnp.float32),
    )(partials)
    return out
